# baseline (speedup 1.0000x reference)
import functools

import jax
import jax.numpy as jnp
from jax import lax
from jax.experimental import pallas as pl
from jax.experimental.pallas import tpu as pltpu
from jax.experimental.pallas import tpu_sc as plsc

B = 16384
EMB = 32
NC = 2
NS = 16
NW = NC * NS
BPW = B // NW
CHUNK = 128
NCHUNK = BPW // CHUNK


def _sc_gather(uid3, iid3, gu_t, gi_t, mu_t, mi_t):
    mesh = plsc.VectorSubcoreMesh(core_axis_name="c", subcore_axis_name="s")
    out_t = [jax.ShapeDtypeStruct((B, EMB), jnp.float32)] * 4

    @functools.partial(
        pl.kernel,
        out_type=out_t,
        mesh=mesh,
        compiler_params=pltpu.CompilerParams(use_tc_tiling_on_sc=False),
        scratch_types=[
            pltpu.VMEM((NCHUNK, CHUNK), jnp.int32),
            pltpu.VMEM((NCHUNK, CHUNK), jnp.int32),
            pltpu.VMEM((BPW, EMB), jnp.float32),
            pltpu.VMEM((BPW, EMB), jnp.float32),
            pltpu.VMEM((BPW, EMB), jnp.float32),
            pltpu.VMEM((BPW, EMB), jnp.float32),
            pltpu.SemaphoreType.DMA,
        ],
    )
    def k(uid_h, iid_h, gu_h, gi_h, mu_h, mi_h,
          gu_o, gi_o, mu_o, mi_o,
          uv, iv, gub, gib, mub, mib, sem):
        wid = lax.axis_index("s") * NC + lax.axis_index("c")
        base = wid * BPW
        pltpu.sync_copy(uid_h.at[wid], uv)
        pltpu.sync_copy(iid_h.at[wid], iv)
        copies = []
        for j in range(NCHUNK):
            dst = pl.ds(j * CHUNK, CHUNK)
            copies.append(pltpu.async_copy(gu_h.at[uv.at[j]], gub.at[dst], sem))
            copies.append(pltpu.async_copy(mu_h.at[uv.at[j]], mub.at[dst], sem))
            copies.append(pltpu.async_copy(gi_h.at[iv.at[j]], gib.at[dst], sem))
            copies.append(pltpu.async_copy(mi_h.at[iv.at[j]], mib.at[dst], sem))
        for c in copies:
            c.wait()
        out_sl = pl.ds(base, BPW)
        pltpu.sync_copy(gub, gu_o.at[out_sl])
        pltpu.sync_copy(gib, gi_o.at[out_sl])
        pltpu.sync_copy(mub, mu_o.at[out_sl])
        pltpu.sync_copy(mib, mi_o.at[out_sl])

    return k(uid3, iid3, gu_t, gi_t, mu_t, mi_t)


BLK = 2048


def _dense_body(gu_r, gi_r, mu_r, mi_r, w1u_r, w1i_r, b1_r, w2_r, b2_r,
                w3_r, b3_r, wog_r, wom_r, bo_r, out_r):
    hp = lax.Precision.HIGHEST
    f32 = jnp.float32
    h = jnp.dot(mu_r[...], w1u_r[...], precision=hp, preferred_element_type=f32)
    h = h + jnp.dot(mi_r[...], w1i_r[...], precision=hp, preferred_element_type=f32)
    h = jnp.maximum(h + b1_r[...], 0.0)
    h = jnp.maximum(
        jnp.dot(h, w2_r[...], precision=hp, preferred_element_type=f32) + b2_r[...], 0.0)
    h = jnp.maximum(
        jnp.dot(h, w3_r[...], precision=hp, preferred_element_type=f32) + b3_r[...], 0.0)
    gmf = gu_r[...] * gi_r[...]
    logit = (jnp.sum(gmf * wog_r[...], axis=1)
             + jnp.sum(h * wom_r[...], axis=1) + bo_r[...])
    out_r[...] = 1.0 / (1.0 + jnp.exp(-logit))


def _tc_dense(gu, gi, mu, mi, w1u, w1i, b1, w2, b2, w3, b3, wog, wom, bo):
    grid = (B // BLK,)
    row_spec = pl.BlockSpec((BLK, EMB), lambda i: (i, 0))

    def full(shape):
        return pl.BlockSpec(shape, lambda i: tuple(0 for _ in shape))

    return pl.pallas_call(
        _dense_body,
        grid=grid,
        in_specs=[
            row_spec, row_spec, row_spec, row_spec,
            full(w1u.shape), full(w1i.shape), full(b1.shape),
            full(w2.shape), full(b2.shape),
            full(w3.shape), full(b3.shape),
            full(wog.shape), full(wom.shape), full(bo.shape),
        ],
        out_specs=pl.BlockSpec((BLK,), lambda i: (i,)),
        out_shape=jax.ShapeDtypeStruct((B,), jnp.float32),
    )(gu, gi, mu, mi, w1u, w1i, b1, w2, b2, w3, b3, wog, wom, bo)


def kernel(user_ids, item_ids, gmf_user_emb, gmf_item_emb, mlp_user_emb,
           mlp_item_emb, W1, b1, W2, b2, W3, b3, Wo, bo):
    uid3 = user_ids.astype(jnp.int32).reshape(NW, NCHUNK, CHUNK)
    iid3 = item_ids.astype(jnp.int32).reshape(NW, NCHUNK, CHUNK)
    gu, gi, mu, mi = _sc_gather(uid3, iid3, gmf_user_emb, gmf_item_emb,
                                mlp_user_emb, mlp_item_emb)
    w1u = W1[:, :EMB].T
    w1i = W1[:, EMB:].T
    w2t = W2.T
    w3t = W3.T
    wog = Wo[:, :EMB]
    wom = Wo[:, EMB:]
    return _tc_dense(gu, gi, mu, mi, w1u, w1i, b1.reshape(1, -1),
                     w2t, b2.reshape(1, -1), w3t, b3.reshape(1, -1),
                     wog, wom, bo)


# speedup vs baseline: 1.6600x; 1.6600x over previous
import functools

import jax
import jax.numpy as jnp
from jax import lax
from jax.experimental import pallas as pl
from jax.experimental.pallas import tpu as pltpu
from jax.experimental.pallas import tpu_sc as plsc

B = 16384
EMB = 32
V = 1000000
NC = 2
NS = 16
NW = NC * NS
BPW = B // NW
CHUNK = 128
NCHUNK = BPW // CHUNK
NPASS = 2
CPP = NCHUNK // NPASS

CB = 4096


def _repack_body(gu_r, mu_r, gi_r, mi_r, out_r):
    for t, r in enumerate((gu_r, mu_r, gi_r, mi_r)):
        out_r[:, t * EMB:(t + 1) * EMB] = r[...].T


def _tc_repack(gu_t, mu_t, gi_t, mi_t):
    grid = (pl.cdiv(V, CB),)
    in_spec = pl.BlockSpec((EMB, CB), lambda i: (0, i))
    return pl.pallas_call(
        _repack_body,
        grid=grid,
        in_specs=[in_spec] * 4,
        out_specs=pl.BlockSpec((CB, 4 * EMB), lambda i: (i, 0)),
        out_shape=jax.ShapeDtypeStruct((V, 4 * EMB), jnp.float32),
    )(gu_t, mu_t, gi_t, mi_t)


def _sc_gather(uid3, iid3, combo):
    mesh = plsc.VectorSubcoreMesh(core_axis_name="c", subcore_axis_name="s")
    out_t = [jax.ShapeDtypeStruct((B, 4 * EMB), jnp.float32)] * 2
    rows_pp = CPP * CHUNK

    @functools.partial(
        pl.kernel,
        out_type=out_t,
        mesh=mesh,
        scratch_types=[
            pltpu.VMEM((NCHUNK, CHUNK), jnp.int32),
            pltpu.VMEM((NCHUNK, CHUNK), jnp.int32),
            pltpu.VMEM((rows_pp, 4 * EMB), jnp.float32),
            pltpu.VMEM((rows_pp, 4 * EMB), jnp.float32),
            pltpu.SemaphoreType.DMA,
        ],
    )
    def k(uid_h, iid_h, combo_h, u_o, i_o, uv, iv, ubuf, ibuf, sem):
        wid = lax.axis_index("s") * NC + lax.axis_index("c")
        base = wid * BPW
        pltpu.sync_copy(uid_h.at[wid], uv)
        pltpu.sync_copy(iid_h.at[wid], iv)
        for p in range(NPASS):
            copies = []
            for j in range(CPP):
                dst = pl.ds(j * CHUNK, CHUNK)
                copies.append(pltpu.async_copy(
                    combo_h.at[uv.at[p * CPP + j]], ubuf.at[dst], sem))
                copies.append(pltpu.async_copy(
                    combo_h.at[iv.at[p * CPP + j]], ibuf.at[dst], sem))
            for c in copies:
                c.wait()
            out_sl = pl.ds(base + p * rows_pp, rows_pp)
            pltpu.sync_copy(ubuf, u_o.at[out_sl])
            pltpu.sync_copy(ibuf, i_o.at[out_sl])

    return k(uid3, iid3, combo)


BLK = 2048


def _dense_body(u_r, i_r, w1u_r, w1i_r, b1_r, w2_r, b2_r,
                w3_r, b3_r, wog_r, wom_r, bo_r, out_r):
    hp = lax.Precision.HIGHEST
    f32 = jnp.float32
    gu = u_r[:, 0:EMB]
    mu = u_r[:, EMB:2 * EMB]
    gi = i_r[:, 2 * EMB:3 * EMB]
    mi = i_r[:, 3 * EMB:4 * EMB]
    h = jnp.dot(mu, w1u_r[...], precision=hp, preferred_element_type=f32)
    h = h + jnp.dot(mi, w1i_r[...], precision=hp, preferred_element_type=f32)
    h = jnp.maximum(h + b1_r[...], 0.0)
    h = jnp.maximum(
        jnp.dot(h, w2_r[...], precision=hp, preferred_element_type=f32) + b2_r[...], 0.0)
    h = jnp.maximum(
        jnp.dot(h, w3_r[...], precision=hp, preferred_element_type=f32) + b3_r[...], 0.0)
    gmf = gu * gi
    logit = (jnp.sum(gmf * wog_r[...], axis=1)
             + jnp.sum(h * wom_r[...], axis=1) + bo_r[...])
    out_r[...] = 1.0 / (1.0 + jnp.exp(-logit))


def _tc_dense(u_rows, i_rows, w1u, w1i, b1, w2, b2, w3, b3, wog, wom, bo):
    grid = (B // BLK,)
    row_spec = pl.BlockSpec((BLK, 4 * EMB), lambda i: (i, 0))

    def full(shape):
        return pl.BlockSpec(shape, lambda i: tuple(0 for _ in shape))

    return pl.pallas_call(
        _dense_body,
        grid=grid,
        in_specs=[
            row_spec, row_spec,
            full(w1u.shape), full(w1i.shape), full(b1.shape),
            full(w2.shape), full(b2.shape),
            full(w3.shape), full(b3.shape),
            full(wog.shape), full(wom.shape), full(bo.shape),
        ],
        out_specs=pl.BlockSpec((BLK,), lambda i: (i,)),
        out_shape=jax.ShapeDtypeStruct((B,), jnp.float32),
    )(u_rows, i_rows, w1u, w1i, b1, w2, b2, w3, b3, wog, wom, bo)


def kernel(user_ids, item_ids, gmf_user_emb, gmf_item_emb, mlp_user_emb,
           mlp_item_emb, W1, b1, W2, b2, W3, b3, Wo, bo):
    uid3 = user_ids.astype(jnp.int32).reshape(NW, NCHUNK, CHUNK)
    iid3 = item_ids.astype(jnp.int32).reshape(NW, NCHUNK, CHUNK)
    combo = _tc_repack(gmf_user_emb.T, mlp_user_emb.T,
                       gmf_item_emb.T, mlp_item_emb.T)
    u_rows, i_rows = _sc_gather(uid3, iid3, combo)
    w1u = W1[:, :EMB].T
    w1i = W1[:, EMB:].T
    wog = Wo[:, :EMB]
    wom = Wo[:, EMB:]
    return _tc_dense(u_rows, i_rows, w1u, w1i, b1.reshape(1, -1),
                     W2.T, b2.reshape(1, -1), W3.T, b3.reshape(1, -1),
                     wog, wom, bo)
